# baseline (speedup 1.0000x reference)
import functools

import jax
import jax.numpy as jnp
from jax import lax
from jax.experimental import pallas as pl
from jax.experimental.pallas import tpu as pltpu
from jax.experimental.pallas import tpu_sc as plsc

_K = 20
_NEG = -1e30


def _stage1_kernel(x_ref, w1t_ref, wdt_ref, b_ref,
                   idx_ref, ut_ref, vt_ref, ssum_ref, ssq_ref,
                   xbt_scr, ut_scr, usq_scr, vt_scr,
                   *, tile_n: int, n: int, k: int, o: int):
    ti = pl.program_id(0)

    @pl.when(ti == 0)
    def _per_batch():
        ssum_ref[...] = jnp.zeros_like(ssum_ref)
        ssq_ref[...] = jnp.zeros_like(ssq_ref)
        xbt = x_ref[0].T
        xbt_scr[...] = xbt
        ut = jnp.dot(xbt, w1t_ref[...], preferred_element_type=jnp.float32)
        ut_scr[...] = ut
        usq_scr[...] = ut * ut
        vt_scr[...] = (jnp.dot(xbt, wdt_ref[...],
                               preferred_element_type=jnp.float32)
                       + b_ref[...])

    xb = x_ref[0]
    xt = xbt_scr[pl.ds(ti * tile_n, tile_n), :]
    xx = jnp.sum(xb * xb, axis=0, keepdims=True)
    p0 = 2.0 * jnp.dot(xt, xb, preferred_element_type=jnp.float32) - xx
    iota = lax.broadcasted_iota(jnp.int32, (tile_n, n), 1)
    iota_k = lax.broadcasted_iota(jnp.int32, (tile_n, k), 1)

    p = p0
    idx_acc = jnp.zeros((tile_n, k), jnp.int32)
    for s in range(k):
        m = jnp.max(p, axis=1, keepdims=True)
        eq = p == m
        jsel = jnp.min(jnp.where(eq, iota, n), axis=1, keepdims=True)
        idx_acc = jnp.where(iota_k == s, jsel, idx_acc)
        p = jnp.where(iota == jsel, _NEG, p)

    idx_ref[0] = idx_acc
    msk = (p != p0).astype(jnp.float32)
    su = jnp.dot(msk, ut_scr[...], preferred_element_type=jnp.float32)
    sq = jnp.dot(msk, usq_scr[...], preferred_element_type=jnp.float32)
    vt = vt_scr[pl.ds(ti * tile_n, tile_n), :]
    ut_ref[0] = jnp.concatenate([ut_scr[pl.ds(ti * tile_n, tile_n), :], vt],
                                axis=1)
    vt_ref[0] = vt
    kf = float(k)
    ssum_ref[0:1, :] += jnp.sum(su + kf * vt, axis=0, keepdims=True)
    ssq_ref[0:1, :] += jnp.sum(sq + 2.0 * vt * su + kf * (vt * vt),
                               axis=0, keepdims=True)


def _make_stage1(B, C, N, O, k, tile_n):
    return pl.pallas_call(
        functools.partial(_stage1_kernel, tile_n=tile_n, n=N, k=k, o=O),
        grid=(N // tile_n,),
        in_specs=[
            pl.BlockSpec((1, C, N), lambda tt: (0, 0, 0)),
            pl.BlockSpec((C, O), lambda tt: (0, 0)),
            pl.BlockSpec((C, O), lambda tt: (0, 0)),
            pl.BlockSpec((1, O), lambda tt: (0, 0)),
        ],
        out_specs=[
            pl.BlockSpec((1, tile_n, k), lambda tt: (0, tt, 0)),
            pl.BlockSpec((1, tile_n, 2 * O), lambda tt: (0, tt, 0)),
            pl.BlockSpec((1, tile_n, O), lambda tt: (0, tt, 0)),
            pl.BlockSpec((8, O), lambda tt: (0, 0)),
            pl.BlockSpec((8, O), lambda tt: (0, 0)),
        ],
        out_shape=[
            jax.ShapeDtypeStruct((1, N, k), jnp.int32),
            jax.ShapeDtypeStruct((1, N, 2 * O), jnp.float32),
            jax.ShapeDtypeStruct((1, N, O), jnp.float32),
            jax.ShapeDtypeStruct((8, O), jnp.float32),
            jax.ShapeDtypeStruct((8, O), jnp.float32),
        ],
        scratch_shapes=[
            pltpu.VMEM((N, C), jnp.float32),
            pltpu.VMEM((N, O), jnp.float32),
            pltpu.VMEM((N, O), jnp.float32),
            pltpu.VMEM((N, O), jnp.float32),
        ],
    )


def _sc_gather_reduce(ut_all, gidx, *, rows: int, o: int, k: int):
    info = plsc.get_sparse_core_info()
    nw = info.num_cores * info.num_subcores
    nodes_per_w = rows // nw
    cn = 4
    n_chunks = nodes_per_w // cn
    nvec = o // 16
    mesh = plsc.VectorSubcoreMesh(core_axis_name="c", subcore_axis_name="s")

    @functools.partial(
        pl.kernel, mesh=mesh,
        out_type=[
            jax.ShapeDtypeStruct((rows, o), jnp.float32),
            jax.ShapeDtypeStruct((rows, o), jnp.float32),
        ],
        scratch_types=[
            pltpu.VMEM((cn * k,), jnp.int32),
            pltpu.VMEM((cn * k,), jnp.int32),
            pltpu.VMEM((cn * k, 2 * o), jnp.float32),
            pltpu.VMEM((cn * k, 2 * o), jnp.float32),
            pltpu.VMEM((cn, o), jnp.float32),
            pltpu.VMEM((cn, o), jnp.float32),
            pltpu.SemaphoreType.DMA,
            pltpu.SemaphoreType.DMA,
        ],
    )
    def sc_kernel(ut_hbm, gidx_hbm, mu_hbm, mn_hbm,
                  idx0, idx1, rows0, rows1, mu_v, mn_v, s0, s1):
        wid = lax.axis_index("s") * info.num_cores + lax.axis_index("c")
        base_w = wid * nodes_per_w

        def reduce_store(rows_v, base_node):
            for node in range(cn):
                for ch in range(nvec):
                    sl = pl.ds(ch * 16, 16)
                    mx = rows_v[node * k, sl]
                    mn = mx
                    for r in range(1, k):
                        val = rows_v[node * k + r, sl]
                        mx = jnp.maximum(mx, val)
                        mn = jnp.minimum(mn, val)
                    mu_v[node, sl] = mx
                    mn_v[node, sl] = mn
            pltpu.sync_copy(mu_v, mu_hbm.at[pl.ds(base_node, cn)])
            pltpu.sync_copy(mn_v, mn_hbm.at[pl.ds(base_node, cn)])

        pltpu.sync_copy(gidx_hbm.at[pl.ds(base_w * k, cn * k)], idx0)
        pltpu.async_copy(ut_hbm.at[idx0], rows0, s0)

        def body(c, _):
            b0 = base_w + (2 * c) * cn
            b1 = base_w + (2 * c + 1) * cn
            pltpu.sync_copy(gidx_hbm.at[pl.ds(b1 * k, cn * k)], idx1)
            pltpu.async_copy(ut_hbm.at[idx1], rows1, s1)
            pltpu.make_async_copy(ut_hbm.at[idx0], rows0, s0).wait()
            reduce_store(rows0, b0)

            @pl.when(c < n_chunks // 2 - 1)
            def _prefetch_next():
                b2 = base_w + (2 * c + 2) * cn
                pltpu.sync_copy(gidx_hbm.at[pl.ds(b2 * k, cn * k)], idx0)
                pltpu.async_copy(ut_hbm.at[idx0], rows0, s0)

            pltpu.make_async_copy(ut_hbm.at[idx1], rows1, s1).wait()
            reduce_store(rows1, b1)
            return ()

        lax.fori_loop(0, n_chunks // 2, body, ())

    return sc_kernel(ut_all, gidx)


def _stage2_kernel(mu_ref, mn_ref, vt_ref, ssum_ref, ssq_ref, g_ref, be_ref,
                   o_ref, *, count: float, eps: float):
    mean = jnp.sum(ssum_ref[...], axis=0, keepdims=True) / count
    var = jnp.sum(ssq_ref[...], axis=0, keepdims=True) / count - mean * mean
    scale = g_ref[...] * lax.rsqrt(var + eps)
    shift = be_ref[...] - mean * scale
    sel = jnp.where(scale > 0.0, mu_ref[0], mn_ref[0]) + vt_ref[0]
    z = sel * scale + shift
    z = jnp.where(z > 0.0, z, 0.2 * z)
    o_ref[0] = z.T


def kernel(x, W, b, gamma, beta):
    B, C, N = x.shape
    O = W.shape[0]
    k = _K
    tile_n = 128 if N % 128 == 0 else N

    w1t = W[:, :C].T
    wdt = (W[:, C:] - W[:, :C]).T
    b2 = b.reshape(1, O)

    stage1 = _make_stage1(B, C, N, O, k, tile_n)
    mus, mns, vts, ssums, ssqs = [], [], [], [], []
    for bi in range(B):
        gidx, utv, vt_b, ssum_p, ssq_p = stage1(
            lax.slice_in_dim(x, bi, bi + 1, axis=0), w1t, wdt, b2)
        mu_b, mn_b = _sc_gather_reduce(utv.reshape(N, 2 * O),
                                       gidx.reshape(N * k),
                                       rows=N, o=O, k=k)
        mus.append(mu_b)
        mns.append(mn_b)
        vts.append(vt_b)
        ssums.append(ssum_p)
        ssqs.append(ssq_p)

    mu = jnp.stack(mus).reshape(B, N, O)
    mn = jnp.stack(mns).reshape(B, N, O)
    vt = jnp.concatenate(vts, axis=0)
    ssum = jnp.concatenate(ssums, axis=0)
    ssq = jnp.concatenate(ssqs, axis=0)

    out = pl.pallas_call(
        functools.partial(_stage2_kernel, count=float(B * N * k), eps=1e-5),
        grid=(B,),
        in_specs=[
            pl.BlockSpec((1, N, O), lambda bb: (bb, 0, 0)),
            pl.BlockSpec((1, N, O), lambda bb: (bb, 0, 0)),
            pl.BlockSpec((1, N, O), lambda bb: (bb, 0, 0)),
            pl.BlockSpec((8 * B, O), lambda bb: (0, 0)),
            pl.BlockSpec((8 * B, O), lambda bb: (0, 0)),
            pl.BlockSpec((1, O), lambda bb: (0, 0)),
            pl.BlockSpec((1, O), lambda bb: (0, 0)),
        ],
        out_specs=pl.BlockSpec((1, O, N), lambda bb: (bb, 0, 0)),
        out_shape=jax.ShapeDtypeStruct((B, O, N), jnp.float32),
    )(mu, mn, vt, ssum, ssq, gamma.reshape(1, O), beta.reshape(1, O))
    return out


# speedup vs baseline: 1.1619x; 1.1619x over previous
import functools

import jax
import jax.numpy as jnp
from jax import lax
from jax.experimental import pallas as pl
from jax.experimental.pallas import tpu as pltpu
from jax.experimental.pallas import tpu_sc as plsc

_K = 20
_NEG = -1e30


def _stage1_kernel(x_ref, w1t_ref, wdt_ref, b_ref,
                   idx_ref, ut_ref, vt_ref, ssum_ref, ssq_ref,
                   xbt_scr, ut_scr, usq_scr, vt_scr,
                   *, tile_n: int, n: int, k: int, o: int):
    ti = pl.program_id(0)

    @pl.when(ti == 0)
    def _per_batch():
        ssum_ref[...] = jnp.zeros_like(ssum_ref)
        ssq_ref[...] = jnp.zeros_like(ssq_ref)
        xbt = x_ref[0].T
        xbt_scr[...] = xbt
        ut = jnp.dot(xbt, w1t_ref[...], preferred_element_type=jnp.float32)
        ut_scr[...] = ut
        usq_scr[...] = ut * ut
        vt_scr[...] = (jnp.dot(xbt, wdt_ref[...],
                               preferred_element_type=jnp.float32)
                       + b_ref[...])

    xb = x_ref[0]
    xt = xbt_scr[pl.ds(ti * tile_n, tile_n), :]
    xx = jnp.sum(xb * xb, axis=0, keepdims=True)
    p0 = 2.0 * jnp.dot(xt, xb, preferred_element_type=jnp.float32) - xx
    iota = lax.broadcasted_iota(jnp.int32, (tile_n, n), 1)
    iota_k = lax.broadcasted_iota(jnp.int32, (tile_n, k), 1)

    p = p0
    idx_acc = jnp.zeros((tile_n, k), jnp.int32)
    for s in range(k):
        m = jnp.max(p, axis=1, keepdims=True)
        eq = p == m
        jsel = jnp.min(jnp.where(eq, iota, n), axis=1, keepdims=True)
        idx_acc = jnp.where(iota_k == s, jsel, idx_acc)
        p = jnp.where(iota == jsel, _NEG, p)

    idx_ref[0] = idx_acc
    msk = (p != p0).astype(jnp.float32)
    su = jnp.dot(msk, ut_scr[...], preferred_element_type=jnp.float32)
    sq = jnp.dot(msk, usq_scr[...], preferred_element_type=jnp.float32)
    vt = vt_scr[pl.ds(ti * tile_n, tile_n), :]
    ut_ref[0] = jnp.concatenate([ut_scr[pl.ds(ti * tile_n, tile_n), :], vt],
                                axis=1)
    vt_ref[0] = vt
    kf = float(k)
    ssum_ref[0:1, :] += jnp.sum(su + kf * vt, axis=0, keepdims=True)
    ssq_ref[0:1, :] += jnp.sum(sq + 2.0 * vt * su + kf * (vt * vt),
                               axis=0, keepdims=True)


def _make_stage1(B, C, N, O, k, tile_n):
    return pl.pallas_call(
        functools.partial(_stage1_kernel, tile_n=tile_n, n=N, k=k, o=O),
        grid=(N // tile_n,),
        in_specs=[
            pl.BlockSpec((1, C, N), lambda tt: (0, 0, 0)),
            pl.BlockSpec((C, O), lambda tt: (0, 0)),
            pl.BlockSpec((C, O), lambda tt: (0, 0)),
            pl.BlockSpec((1, O), lambda tt: (0, 0)),
        ],
        out_specs=[
            pl.BlockSpec((1, tile_n, k), lambda tt: (0, tt, 0)),
            pl.BlockSpec((1, tile_n, 2 * O), lambda tt: (0, tt, 0)),
            pl.BlockSpec((1, tile_n, O), lambda tt: (0, tt, 0)),
            pl.BlockSpec((8, O), lambda tt: (0, 0)),
            pl.BlockSpec((8, O), lambda tt: (0, 0)),
        ],
        out_shape=[
            jax.ShapeDtypeStruct((1, N, k), jnp.int32),
            jax.ShapeDtypeStruct((1, N, 2 * O), jnp.float32),
            jax.ShapeDtypeStruct((1, N, O), jnp.float32),
            jax.ShapeDtypeStruct((8, O), jnp.float32),
            jax.ShapeDtypeStruct((8, O), jnp.float32),
        ],
        scratch_shapes=[
            pltpu.VMEM((N, C), jnp.float32),
            pltpu.VMEM((N, O), jnp.float32),
            pltpu.VMEM((N, O), jnp.float32),
            pltpu.VMEM((N, O), jnp.float32),
        ],
    )


def _sc_gather_reduce(ut_all, gidx, *, rows: int, o: int, k: int):
    info = plsc.get_sparse_core_info()
    nw = info.num_cores * info.num_subcores
    nodes_per_w = rows // nw
    cn = 4
    n_chunks = nodes_per_w // cn
    nvec = o // 16
    mesh = plsc.VectorSubcoreMesh(core_axis_name="c", subcore_axis_name="s")

    @functools.partial(
        pl.kernel, mesh=mesh,
        out_type=[
            jax.ShapeDtypeStruct((rows, o), jnp.float32),
            jax.ShapeDtypeStruct((rows, o), jnp.float32),
        ],
        scratch_types=[
            pltpu.VMEM((cn * k,), jnp.int32),
            pltpu.VMEM((cn * k,), jnp.int32),
            pltpu.VMEM((cn * k, 2 * o), jnp.float32),
            pltpu.VMEM((cn * k, 2 * o), jnp.float32),
            pltpu.VMEM((cn, o), jnp.float32),
            pltpu.VMEM((cn, o), jnp.float32),
            pltpu.SemaphoreType.DMA,
            pltpu.SemaphoreType.DMA,
        ],
    )
    def sc_kernel(ut_hbm, gidx_hbm, mu_hbm, mn_hbm,
                  idx0, idx1, rows0, rows1, mu_v, mn_v, s0, s1):
        wid = lax.axis_index("s") * info.num_cores + lax.axis_index("c")
        base_w = wid * nodes_per_w

        def reduce_store(rows_v, base_node):
            for node in range(cn):
                for ch in range(nvec):
                    sl = pl.ds(ch * 16, 16)
                    mx = rows_v[node * k, sl]
                    mn = mx
                    for r in range(1, k):
                        val = rows_v[node * k + r, sl]
                        mx = jnp.maximum(mx, val)
                        mn = jnp.minimum(mn, val)
                    mu_v[node, sl] = mx
                    mn_v[node, sl] = mn
            pltpu.sync_copy(mu_v, mu_hbm.at[pl.ds(base_node, cn)])
            pltpu.sync_copy(mn_v, mn_hbm.at[pl.ds(base_node, cn)])

        pltpu.sync_copy(gidx_hbm.at[pl.ds(base_w * k, cn * k)], idx0)
        pltpu.async_copy(ut_hbm.at[idx0], rows0, s0)

        def body(c, _):
            b0 = base_w + (2 * c) * cn
            b1 = base_w + (2 * c + 1) * cn
            pltpu.sync_copy(gidx_hbm.at[pl.ds(b1 * k, cn * k)], idx1)
            pltpu.async_copy(ut_hbm.at[idx1], rows1, s1)
            pltpu.make_async_copy(ut_hbm.at[idx0], rows0, s0).wait()
            reduce_store(rows0, b0)

            @pl.when(c < n_chunks // 2 - 1)
            def _prefetch_next():
                b2 = base_w + (2 * c + 2) * cn
                pltpu.sync_copy(gidx_hbm.at[pl.ds(b2 * k, cn * k)], idx0)
                pltpu.async_copy(ut_hbm.at[idx0], rows0, s0)

            pltpu.make_async_copy(ut_hbm.at[idx1], rows1, s1).wait()
            reduce_store(rows1, b1)
            return ()

        lax.fori_loop(0, n_chunks // 2, body, ())

    return sc_kernel(ut_all, gidx)


def _stage2_kernel(mu_ref, mn_ref, vt_ref, ssum_ref, ssq_ref, g_ref, be_ref,
                   o_ref, *, count: float, eps: float):
    mean = jnp.sum(ssum_ref[...], axis=0, keepdims=True) / count
    var = jnp.sum(ssq_ref[...], axis=0, keepdims=True) / count - mean * mean
    scale = g_ref[...] * lax.rsqrt(var + eps)
    shift = be_ref[...] - mean * scale
    sel = jnp.where(scale > 0.0, mu_ref[0], mn_ref[0]) + vt_ref[0]
    z = sel * scale + shift
    z = jnp.where(z > 0.0, z, 0.2 * z)
    o_ref[0] = z.T


def kernel(x, W, b, gamma, beta):
    B, C, N = x.shape
    O = W.shape[0]
    k = _K
    tile_n = 256 if N % 256 == 0 else N

    w1t = W[:, :C].T
    wdt = (W[:, C:] - W[:, :C]).T
    b2 = b.reshape(1, O)

    stage1 = _make_stage1(B, C, N, O, k, tile_n)
    mus, mns, vts, ssums, ssqs = [], [], [], [], []
    for bi in range(B):
        gidx, utv, vt_b, ssum_p, ssq_p = stage1(
            lax.slice_in_dim(x, bi, bi + 1, axis=0), w1t, wdt, b2)
        mu_b, mn_b = _sc_gather_reduce(utv.reshape(N, 2 * O),
                                       gidx.reshape(N * k),
                                       rows=N, o=O, k=k)
        mus.append(mu_b)
        mns.append(mn_b)
        vts.append(vt_b)
        ssums.append(ssum_p)
        ssqs.append(ssq_p)

    mu = jnp.stack(mus).reshape(B, N, O)
    mn = jnp.stack(mns).reshape(B, N, O)
    vt = jnp.concatenate(vts, axis=0)
    ssum = jnp.concatenate(ssums, axis=0)
    ssq = jnp.concatenate(ssqs, axis=0)

    out = pl.pallas_call(
        functools.partial(_stage2_kernel, count=float(B * N * k), eps=1e-5),
        grid=(B,),
        in_specs=[
            pl.BlockSpec((1, N, O), lambda bb: (bb, 0, 0)),
            pl.BlockSpec((1, N, O), lambda bb: (bb, 0, 0)),
            pl.BlockSpec((1, N, O), lambda bb: (bb, 0, 0)),
            pl.BlockSpec((8 * B, O), lambda bb: (0, 0)),
            pl.BlockSpec((8 * B, O), lambda bb: (0, 0)),
            pl.BlockSpec((1, O), lambda bb: (0, 0)),
            pl.BlockSpec((1, O), lambda bb: (0, 0)),
        ],
        out_specs=pl.BlockSpec((1, O, N), lambda bb: (bb, 0, 0)),
        out_shape=jax.ShapeDtypeStruct((B, O, N), jnp.float32),
    )(mu, mn, vt, ssum, ssq, gamma.reshape(1, O), beta.reshape(1, O))
    return out
